# baseline (speedup 1.0000x reference)
import functools

import jax
import jax.numpy as jnp
from jax import lax
from jax.experimental import pallas as pl
from jax.experimental.pallas import tpu as pltpu
from jax.experimental.pallas import tpu_sc as plsc

NC = 2
NS = 16
NW = NC * NS
CHUNK = 128
BATCH = 8


@functools.partial(jax.jit, static_argnames=("nchunks", "d"))
def _sc_gather(idx3, tab2d, *, nchunks, d):
    rows_total = NW * nchunks * CHUNK
    dt = tab2d.dtype
    nbatch, btail = divmod(nchunks, BATCH)
    mesh = plsc.VectorSubcoreMesh(
        core_axis_name="c", subcore_axis_name="s", num_cores=NC, num_subcores=NS
    )

    @functools.partial(
        pl.kernel,
        mesh=mesh,
        compiler_params=pltpu.CompilerParams(use_tc_tiling_on_sc=False),
        out_type=jax.ShapeDtypeStruct((rows_total, d), dt),
        scratch_types=[
            pltpu.VMEM((nchunks, CHUNK), jnp.int32),
            pltpu.VMEM((BATCH * CHUNK, d), dt),
            pltpu.VMEM((BATCH * CHUNK, d), dt),
            pltpu.SemaphoreType.DMA,
            pltpu.SemaphoreType.DMA,
            pltpu.SemaphoreType.DMA,
        ],
    )
    def k(idx_hbm, tab2d, out_hbm, idx_v, buf0, buf1, gsem0, gsem1, wsem):
        wid = lax.axis_index("s") * NC + lax.axis_index("c")
        pltpu.sync_copy(idx_hbm.at[wid], idx_v)
        bufs = (buf0, buf1)
        gsems = (gsem0, gsem1)

        def fire(bi, bank):
            descs = []
            for j in range(BATCH):
                descs.append(
                    pltpu.async_copy(
                        tab2d.at[idx_v.at[bi * BATCH + j]],
                        bufs[bank].at[pl.ds(j * CHUNK, CHUNK)],
                        gsems[bank],
                    )
                )
            return descs

        def drain_write(descs, bi, bank):
            for desc in descs:
                desc.wait()
            return pltpu.async_copy(
                bufs[bank],
                out_hbm.at[pl.ds((wid * nchunks + bi * BATCH) * CHUNK, BATCH * CHUNK)],
                wsem,
            )

        @pl.loop(0, nbatch // 2)
        def _body(i):
            bi0 = i * 2
            d0 = fire(bi0, 0)
            d1 = fire(bi0 + 1, 1)
            w0 = drain_write(d0, bi0, 0)
            w1 = drain_write(d1, bi0 + 1, 1)
            w0.wait()
            w1.wait()

        if nbatch % 2:
            bi = nbatch - 1
            d0 = fire(bi, 0)
            drain_write(d0, bi, 0).wait()
        if btail:
            base = nbatch * BATCH
            descs = []
            for j in range(btail):
                descs.append(
                    pltpu.async_copy(
                        tab2d.at[idx_v.at[base + j]],
                        buf0.at[pl.ds(j * CHUNK, CHUNK)],
                        gsem0,
                    )
                )
            for desc in descs:
                desc.wait()
            pltpu.sync_copy(
                buf0.at[pl.ds(0, btail * CHUNK)],
                out_hbm.at[pl.ds((wid * nchunks + base) * CHUNK, btail * CHUNK)],
            )

    return k(idx3, tab2d)


def _mm_bias(a, w, bias, bm):
    m, kdim = a.shape
    nout = w.shape[1]

    def body(a_ref, w_ref, b_ref, o_ref):
        o_ref[...] = (
            jnp.dot(a_ref[...], w_ref[...], preferred_element_type=jnp.float32)
            + b_ref[...]
        )

    return pl.pallas_call(
        body,
        grid=(m // bm,),
        in_specs=[
            pl.BlockSpec((bm, kdim), lambda i: (i, 0)),
            pl.BlockSpec((kdim, nout), lambda i: (0, 0)),
            pl.BlockSpec((1, nout), lambda i: (0, 0)),
        ],
        out_specs=pl.BlockSpec((bm, nout), lambda i: (i, 0)),
        out_shape=jax.ShapeDtypeStruct((m, nout), jnp.float32),
    )(a, w, bias.reshape(1, nout))


def kernel(x, tables, W, b):
    s, bsz, f = x.shape
    _, v, d = tables.shape
    n = s * bsz * f
    assert n % (NW * CHUNK) == 0
    nchunks = n // (NW * CHUNK)
    flat_idx = (x.astype(jnp.int32) + jnp.arange(f, dtype=jnp.int32) * v).reshape(
        NW, nchunks, CHUNK
    )
    tab16 = tables.astype(jnp.bfloat16).reshape(-1, d)
    rows = _sc_gather(flat_idx, tab16, nchunks=nchunks, d=d)
    a = rows.reshape(s * bsz, f * d)
    y = _mm_bias(a, W.astype(jnp.bfloat16), b, bm=1024)
    return y.reshape(s, bsz, W.shape[1])


# speedup vs baseline: 1.2598x; 1.2598x over previous
import functools

import jax
import jax.numpy as jnp
from jax import lax
from jax.experimental import pallas as pl
from jax.experimental.pallas import tpu as pltpu
from jax.experimental.pallas import tpu_sc as plsc

NC = 2
NS = 16
NW = NC * NS
CHUNK = 128
BATCH = 8


@functools.partial(jax.jit, static_argnames=("nchunks", "d"))
def _sc_gather(idx3, tab2d, *, nchunks, d):
    rows_total = NW * nchunks * CHUNK
    nbatch, btail = divmod(nchunks, BATCH)
    mesh = plsc.VectorSubcoreMesh(
        core_axis_name="c", subcore_axis_name="s", num_cores=NC, num_subcores=NS
    )

    @functools.partial(
        pl.kernel,
        mesh=mesh,
        compiler_params=pltpu.CompilerParams(use_tc_tiling_on_sc=False),
        out_type=jax.ShapeDtypeStruct((rows_total, d), jnp.float32),
        scratch_types=[
            pltpu.VMEM((nchunks, CHUNK), jnp.int32),
            pltpu.VMEM((BATCH * CHUNK, d), jnp.float32),
            pltpu.VMEM((BATCH * CHUNK, d), jnp.float32),
            pltpu.SemaphoreType.DMA,
            pltpu.SemaphoreType.DMA,
            pltpu.SemaphoreType.DMA,
        ],
    )
    def k(idx_hbm, tab2d, out_hbm, idx_v, buf0, buf1, gsem0, gsem1, wsem):
        wid = lax.axis_index("s") * NC + lax.axis_index("c")
        pltpu.sync_copy(idx_hbm.at[wid], idx_v)
        bufs = (buf0, buf1)
        gsems = (gsem0, gsem1)

        def fire(bi, bank):
            descs = []
            for j in range(BATCH):
                descs.append(
                    pltpu.async_copy(
                        tab2d.at[idx_v.at[bi * BATCH + j]],
                        bufs[bank].at[pl.ds(j * CHUNK, CHUNK)],
                        gsems[bank],
                    )
                )
            return descs

        def drain_write(descs, bi, bank):
            for desc in descs:
                desc.wait()
            return pltpu.async_copy(
                bufs[bank],
                out_hbm.at[pl.ds((wid * nchunks + bi * BATCH) * CHUNK, BATCH * CHUNK)],
                wsem,
            )

        @pl.loop(0, nbatch // 2)
        def _body(i):
            bi0 = i * 2
            d0 = fire(bi0, 0)
            d1 = fire(bi0 + 1, 1)
            w0 = drain_write(d0, bi0, 0)
            w1 = drain_write(d1, bi0 + 1, 1)
            w0.wait()
            w1.wait()

        if nbatch % 2:
            bi = nbatch - 1
            d0 = fire(bi, 0)
            drain_write(d0, bi, 0).wait()
        if btail:
            base = nbatch * BATCH
            descs = []
            for j in range(btail):
                descs.append(
                    pltpu.async_copy(
                        tab2d.at[idx_v.at[base + j]],
                        buf0.at[pl.ds(j * CHUNK, CHUNK)],
                        gsem0,
                    )
                )
            for desc in descs:
                desc.wait()
            pltpu.sync_copy(
                buf0.at[pl.ds(0, btail * CHUNK)],
                out_hbm.at[pl.ds((wid * nchunks + base) * CHUNK, btail * CHUNK)],
            )

    return k(idx3, tab2d)


def _mm_bias(a, w, bias, bm):
    m, kdim = a.shape
    nout = w.shape[1]

    def body(a_ref, w_ref, b_ref, o_ref):
        o_ref[...] = (
            jnp.dot(a_ref[...], w_ref[...], preferred_element_type=jnp.float32)
            + b_ref[...]
        )

    return pl.pallas_call(
        body,
        grid=(m // bm,),
        in_specs=[
            pl.BlockSpec((bm, kdim), lambda i: (i, 0)),
            pl.BlockSpec((kdim, nout), lambda i: (0, 0)),
            pl.BlockSpec((1, nout), lambda i: (0, 0)),
        ],
        out_specs=pl.BlockSpec((bm, nout), lambda i: (i, 0)),
        out_shape=jax.ShapeDtypeStruct((m, nout), jnp.float32),
    )(a, w, bias.reshape(1, nout))


def kernel(x, tables, W, b):
    s, bsz, f = x.shape
    _, v, d = tables.shape
    n = s * bsz * f
    assert n % (NW * CHUNK) == 0
    nchunks = n // (NW * CHUNK)
    flat_idx = (x.astype(jnp.int32) + jnp.arange(f, dtype=jnp.int32) * v).reshape(
        NW, nchunks, CHUNK
    )
    rows = _sc_gather(flat_idx, tables.reshape(-1, d), nchunks=nchunks, d=d)
    a = rows.reshape(s * bsz, f * d)
    y = _mm_bias(a, W, b, bm=1024)
    return y.reshape(s, bsz, W.shape[1])


# speedup vs baseline: 1.2631x; 1.0026x over previous
import functools

import jax
import jax.numpy as jnp
from jax import lax
from jax.experimental import pallas as pl
from jax.experimental.pallas import tpu as pltpu
from jax.experimental.pallas import tpu_sc as plsc

NC = 2
NS = 16
NW = NC * NS
CHUNK = 128
BATCH = 10


@functools.partial(jax.jit, static_argnames=("nchunks", "d"))
def _sc_gather(idx3, tab2d, *, nchunks, d):
    rows_total = NW * nchunks * CHUNK
    nbatch, btail = divmod(nchunks, BATCH)
    mesh = plsc.VectorSubcoreMesh(
        core_axis_name="c", subcore_axis_name="s", num_cores=NC, num_subcores=NS
    )

    @functools.partial(
        pl.kernel,
        mesh=mesh,
        compiler_params=pltpu.CompilerParams(use_tc_tiling_on_sc=False),
        out_type=jax.ShapeDtypeStruct((rows_total, d), jnp.float32),
        scratch_types=[
            pltpu.VMEM((nchunks, CHUNK), jnp.int32),
            pltpu.VMEM((BATCH * CHUNK, d), jnp.float32),
            pltpu.VMEM((BATCH * CHUNK, d), jnp.float32),
            pltpu.SemaphoreType.DMA,
            pltpu.SemaphoreType.DMA,
            pltpu.SemaphoreType.DMA,
        ],
    )
    def k(idx_hbm, tab2d, out_hbm, idx_v, buf0, buf1, gsem0, gsem1, wsem):
        wid = lax.axis_index("s") * NC + lax.axis_index("c")
        pltpu.sync_copy(idx_hbm.at[wid], idx_v)
        bufs = (buf0, buf1)
        gsems = (gsem0, gsem1)

        def fire(bi, bank):
            descs = []
            for j in range(BATCH):
                descs.append(
                    pltpu.async_copy(
                        tab2d.at[idx_v.at[bi * BATCH + j]],
                        bufs[bank].at[pl.ds(j * CHUNK, CHUNK)],
                        gsems[bank],
                    )
                )
            return descs

        def drain_write(descs, bi, bank):
            for desc in descs:
                desc.wait()
            return pltpu.async_copy(
                bufs[bank],
                out_hbm.at[pl.ds((wid * nchunks + bi * BATCH) * CHUNK, BATCH * CHUNK)],
                wsem,
            )

        @pl.loop(0, nbatch // 2)
        def _body(i):
            bi0 = i * 2
            d0 = fire(bi0, 0)
            d1 = fire(bi0 + 1, 1)
            w0 = drain_write(d0, bi0, 0)
            w1 = drain_write(d1, bi0 + 1, 1)
            w0.wait()
            w1.wait()

        if nbatch % 2:
            bi = nbatch - 1
            d0 = fire(bi, 0)
            drain_write(d0, bi, 0).wait()
        if btail:
            base = nbatch * BATCH
            descs = []
            for j in range(btail):
                descs.append(
                    pltpu.async_copy(
                        tab2d.at[idx_v.at[base + j]],
                        buf0.at[pl.ds(j * CHUNK, CHUNK)],
                        gsem0,
                    )
                )
            for desc in descs:
                desc.wait()
            pltpu.sync_copy(
                buf0.at[pl.ds(0, btail * CHUNK)],
                out_hbm.at[pl.ds((wid * nchunks + base) * CHUNK, btail * CHUNK)],
            )

    return k(idx3, tab2d)


def _mm_bias(a, w, bias, bm):
    m, kdim = a.shape
    nout = w.shape[1]

    def body(a_ref, w_ref, b_ref, o_ref):
        o_ref[...] = (
            jnp.dot(a_ref[...], w_ref[...], preferred_element_type=jnp.float32)
            + b_ref[...]
        )

    return pl.pallas_call(
        body,
        grid=(m // bm,),
        in_specs=[
            pl.BlockSpec((bm, kdim), lambda i: (i, 0)),
            pl.BlockSpec((kdim, nout), lambda i: (0, 0)),
            pl.BlockSpec((1, nout), lambda i: (0, 0)),
        ],
        out_specs=pl.BlockSpec((bm, nout), lambda i: (i, 0)),
        out_shape=jax.ShapeDtypeStruct((m, nout), jnp.float32),
    )(a, w, bias.reshape(1, nout))


def kernel(x, tables, W, b):
    s, bsz, f = x.shape
    _, v, d = tables.shape
    n = s * bsz * f
    assert n % (NW * CHUNK) == 0
    nchunks = n // (NW * CHUNK)
    flat_idx = (x.astype(jnp.int32) + jnp.arange(f, dtype=jnp.int32) * v).reshape(
        NW, nchunks, CHUNK
    )
    rows = _sc_gather(flat_idx, tables.reshape(-1, d), nchunks=nchunks, d=d)
    a = rows.reshape(s * bsz, f * d)
    y = _mm_bias(a, W, b, bm=1024)
    return y.reshape(s, bsz, W.shape[1])
